# Initial kernel scaffold; baseline (speedup 1.0000x reference)
import jax
import jax.numpy as jnp
from jax.experimental import pallas as pl


def kernel(tokens, table):
    raise NotImplementedError("write your pallas kernel here")


# speedup vs baseline: 1.3067x; 1.3067x over previous
import functools
import math

import jax
import jax.numpy as jnp
from jax import lax
from jax.experimental import pallas as pl
from jax.experimental.pallas import tpu as pltpu
from jax.experimental.pallas import tpu_sc as plsc

_NUM_WORKERS = 32
_CHUNK = 1600
_LANES = 16


def _emb_lookup(flat_idx, table, *, n, d, chunks_per_worker, scale):
    mesh = plsc.VectorSubcoreMesh(core_axis_name="c", subcore_axis_name="s")
    per_worker = n // _NUM_WORKERS

    @functools.partial(
        pl.kernel,
        mesh=mesh,
        out_type=jax.ShapeDtypeStruct((n, d), jnp.float32),
        scratch_types=[
            pltpu.VMEM((_CHUNK,), jnp.int32),
            pltpu.VMEM((_CHUNK, d), jnp.float32),
            pltpu.SemaphoreType.DMA,
        ],
        compiler_params=pltpu.CompilerParams(use_tc_tiling_on_sc=False),
    )
    def body(idx_hbm, table_hbm, out_hbm, idx_v, rows_v, sem):
        wid = lax.axis_index("s") * 2 + lax.axis_index("c")
        base = wid * per_worker

        def chunk_body(ci, carry):
            off = base + ci * _CHUNK
            pltpu.sync_copy(idx_hbm.at[pl.ds(off, _CHUNK)], idx_v)
            pltpu.async_copy(table_hbm.at[idx_v], rows_v, sem).wait()

            def scale_body(r, c2):
                rows_v[r, pl.ds(0, _LANES)] = rows_v[r, pl.ds(0, _LANES)] * scale
                rows_v[r, pl.ds(_LANES, _LANES)] = (
                    rows_v[r, pl.ds(_LANES, _LANES)] * scale
                )
                return c2

            lax.fori_loop(0, _CHUNK, scale_body, 0)
            pltpu.sync_copy(rows_v, out_hbm.at[pl.ds(off, _CHUNK)])
            return carry

        lax.fori_loop(0, chunks_per_worker, chunk_body, 0)

    return body(flat_idx, table)


def kernel(tokens, table):
    b, l = tokens.shape
    v, d = table.shape
    n = b * l
    per_worker = n // _NUM_WORKERS
    flat_idx = tokens.reshape(n).astype(jnp.int32)
    out = _emb_lookup(
        flat_idx,
        table,
        n=n,
        d=d,
        chunks_per_worker=per_worker // _CHUNK,
        scale=math.sqrt(d),
    )
    return out.reshape(b, l, d)


# speedup vs baseline: 1.4772x; 1.1305x over previous
import functools
import math

import jax
import jax.numpy as jnp
from jax import lax
from jax.experimental import pallas as pl
from jax.experimental.pallas import tpu as pltpu
from jax.experimental.pallas import tpu_sc as plsc

_NUM_WORKERS = 32
_CHUNK = 1024
_NBUF = 3
_LANES = 16


def _emb_lookup(flat_idx, table, *, n, d, chunks_per_worker, scale):
    mesh = plsc.VectorSubcoreMesh(core_axis_name="c", subcore_axis_name="s")
    per_worker = n // _NUM_WORKERS
    nc = chunks_per_worker

    @functools.partial(
        pl.kernel,
        mesh=mesh,
        out_type=jax.ShapeDtypeStruct((n, d), jnp.float32),
        scratch_types=[
            [pltpu.VMEM((_CHUNK,), jnp.int32) for _ in range(_NBUF)],
            [pltpu.VMEM((_CHUNK, d), jnp.float32) for _ in range(_NBUF)],
            [pltpu.SemaphoreType.DMA for _ in range(_NBUF)],
            [pltpu.SemaphoreType.DMA for _ in range(_NBUF)],
        ],
        compiler_params=pltpu.CompilerParams(use_tc_tiling_on_sc=False),
    )
    def body(idx_hbm, table_hbm, out_hbm, idx_v, rows_v, gsem, wsem):
        wid = lax.axis_index("s") * 2 + lax.axis_index("c")
        base = wid * per_worker

        def start_gather(ci):
            b = ci % _NBUF
            off = base + ci * _CHUNK
            pltpu.sync_copy(idx_hbm.at[pl.ds(off, _CHUNK)], idx_v[b])
            return pltpu.async_copy(table_hbm.at[idx_v[b]], rows_v[b], gsem[b])

        gd = [None] * _NBUF
        wd = [None] * _NBUF
        gd[0] = start_gather(0)
        if nc > 1:
            gd[1] = start_gather(1)

        for ci in range(nc):
            b = ci % _NBUF
            gd[b].wait()

            def scale_rows(r8, carry, rows=rows_v[b]):
                for k in range(8):
                    r = r8 * 8 + k
                    rows[r, pl.ds(0, _LANES)] = rows[r, pl.ds(0, _LANES)] * scale
                    rows[r, pl.ds(_LANES, _LANES)] = (
                        rows[r, pl.ds(_LANES, _LANES)] * scale
                    )
                return carry

            lax.fori_loop(0, _CHUNK // 8, scale_rows, 0)
            off = base + ci * _CHUNK
            wd[b] = pltpu.async_copy(rows_v[b], out_hbm.at[pl.ds(off, _CHUNK)],
                                     wsem[b])
            nxt = ci + 2
            if nxt < nc:
                nb = nxt % _NBUF
                if wd[nb] is not None:
                    wd[nb].wait()
                gd[nb] = start_gather(nxt)
        for b in range(min(_NBUF, nc)):
            if wd[b] is not None:
                wd[b].wait()

    return body(flat_idx, table)


def kernel(tokens, table):
    b, l = tokens.shape
    v, d = table.shape
    n = b * l
    per_worker = n // _NUM_WORKERS
    flat_idx = tokens.reshape(n).astype(jnp.int32)
    out = _emb_lookup(
        flat_idx,
        table,
        n=n,
        d=d,
        chunks_per_worker=per_worker // _CHUNK,
        scale=math.sqrt(d),
    )
    return out.reshape(b, l, d)
